# Initial kernel scaffold; baseline (speedup 1.0000x reference)
import jax
import jax.numpy as jnp
from jax.experimental import pallas as pl


def kernel(x, edge_index, edge_weights, W, b):
    raise NotImplementedError("write your pallas kernel here")


# speedup vs baseline: 108.7174x; 108.7174x over previous
import functools

import jax
import jax.numpy as jnp
from jax import lax
from jax.experimental import pallas as pl
from jax.experimental.pallas import tpu as pltpu
from jax.experimental.pallas import tpu_sc as plsc

N = 10000
E = 320000
D = 128
C = 1

_INFO = plsc.get_sparse_core_info()
_NC = _INFO.num_cores
_NS = _INFO.num_subcores
NW = _NC * _NS
EPW = E // NW
_L = 16

_MESH = plsc.VectorSubcoreMesh(core_axis_name="c", subcore_axis_name="s")
_SC_PARAMS = pltpu.CompilerParams(needs_layout_passes=False)


def _worker_id():
  return lax.axis_index("s") * _NC + lax.axis_index("c")


def _zero_vmem(ref, n):
  z = jnp.zeros((_L,), jnp.float32)

  def body(i, carry):
    ref[pl.ds(i * _L, _L)] = z
    return carry

  lax.fori_loop(0, n // _L, body, 0)


@functools.partial(
    pl.kernel,
    out_type=jax.ShapeDtypeStruct((NW, N), jnp.float32),
    mesh=_MESH,
    compiler_params=_SC_PARAMS,
    scratch_types=[
        pltpu.VMEM((EPW,), jnp.int32),
        pltpu.VMEM((EPW,), jnp.float32),
        pltpu.VMEM((N,), jnp.float32),
    ],
)
def _sc_degree(col_hbm, w_hbm, out_hbm, col_v, w_v, acc_v):
  wid = _worker_id()
  base = wid * EPW
  pltpu.sync_copy(col_hbm.at[pl.ds(base, EPW)], col_v)
  pltpu.sync_copy(w_hbm.at[pl.ds(base, EPW)], w_v)
  _zero_vmem(acc_v, N)

  def body(i, carry):
    c = col_v[pl.ds(i * _L, _L)]
    wv = w_v[pl.ds(i * _L, _L)]
    plsc.addupdate_scatter(acc_v, [c], wv)
    return carry

  lax.fori_loop(0, EPW // _L, body, 0)
  pltpu.sync_copy(acc_v, out_hbm.at[wid])


@functools.partial(
    pl.kernel,
    out_type=jax.ShapeDtypeStruct((NW, N), jnp.float32),
    mesh=_MESH,
    compiler_params=_SC_PARAMS,
    scratch_types=[
        pltpu.VMEM((EPW,), jnp.int32),
        pltpu.VMEM((EPW,), jnp.int32),
        pltpu.VMEM((EPW,), jnp.float32),
        pltpu.VMEM((N,), jnp.float32),
        pltpu.VMEM((N,), jnp.float32),
    ],
)
def _sc_aggregate(row_hbm, col_hbm, w_hbm, g_hbm, out_hbm,
                  row_v, col_v, w_v, g_v, acc_v):
  wid = _worker_id()
  base = wid * EPW
  pltpu.sync_copy(row_hbm.at[pl.ds(base, EPW)], row_v)
  pltpu.sync_copy(col_hbm.at[pl.ds(base, EPW)], col_v)
  pltpu.sync_copy(w_hbm.at[pl.ds(base, EPW)], w_v)
  pltpu.sync_copy(g_hbm, g_v)
  _zero_vmem(acc_v, N)

  def body(i, carry):
    r = row_v[pl.ds(i * _L, _L)]
    c = col_v[pl.ds(i * _L, _L)]
    wv = w_v[pl.ds(i * _L, _L)]
    gv = plsc.load_gather(g_v, [r])
    plsc.addupdate_scatter(acc_v, [c], wv * gv)
    return carry

  lax.fori_loop(0, EPW // _L, body, 0)
  pltpu.sync_copy(acc_v, out_hbm.at[wid])


def _tc_matvec_body(x_ref, w_ref, h_ref):
  h_ref[...] = jnp.dot(x_ref[...], w_ref[...],
                       preferred_element_type=jnp.float32)


def _tc_matvec(x, w):
  rows = 1000
  return pl.pallas_call(
      _tc_matvec_body,
      grid=(N // rows,),
      in_specs=[
          pl.BlockSpec((rows, D), lambda i: (i, 0)),
          pl.BlockSpec((D, C), lambda i: (0, 0)),
      ],
      out_specs=pl.BlockSpec((rows, C), lambda i: (i, 0)),
      out_shape=jax.ShapeDtypeStruct((N, C), jnp.float32),
  )(x, w)


def _tc_prep_body(parts_ref, h_ref, dis_ref, g_ref):
  deg = jnp.sum(parts_ref[...], axis=0, keepdims=True) + 1.0
  dis = lax.rsqrt(deg)
  dis_ref[...] = dis
  g_ref[...] = dis * h_ref[...]


def _tc_prep(parts, h_row):
  return pl.pallas_call(
      _tc_prep_body,
      out_shape=(
          jax.ShapeDtypeStruct((1, N), jnp.float32),
          jax.ShapeDtypeStruct((1, N), jnp.float32),
      ),
  )(parts, h_row)


def _tc_final_body(parts_ref, dis_ref, g_ref, b_ref, out_ref):
  s = jnp.sum(parts_ref[...], axis=0, keepdims=True)
  out_ref[...] = dis_ref[...] * (s + g_ref[...]) + b_ref[0, 0]


def _tc_final(parts, dis, g, b):
  return pl.pallas_call(
      _tc_final_body,
      out_shape=jax.ShapeDtypeStruct((1, N), jnp.float32),
  )(parts, dis, g, b.reshape(1, 1))


def kernel(x, edge_index, edge_weights, W, b):
  row = edge_index[0]
  col = edge_index[1]

  h = _tc_matvec(x, W)
  deg_parts = _sc_degree(col, edge_weights)
  dis, g = _tc_prep(deg_parts, h.reshape(1, N))
  s_parts = _sc_aggregate(row, col, edge_weights, g.reshape(N))
  out = _tc_final(s_parts, dis, g, b)
  return out.reshape(N, C)


# speedup vs baseline: 147.9941x; 1.3613x over previous
import functools

import jax
import jax.numpy as jnp
from jax import lax
from jax.experimental import pallas as pl
from jax.experimental.pallas import tpu as pltpu
from jax.experimental.pallas import tpu_sc as plsc

N = 10000
E = 320000
D = 128
C = 1

_INFO = plsc.get_sparse_core_info()
_NC = _INFO.num_cores
_NS = _INFO.num_subcores
NW = _NC * _NS
EPW = E // NW
_L = 16

_MESH = plsc.VectorSubcoreMesh(core_axis_name="c", subcore_axis_name="s")
_SC_PARAMS = pltpu.CompilerParams(needs_layout_passes=False)


def _worker_id():
  return lax.axis_index("s") * _NC + lax.axis_index("c")


def _zero_vmem(ref, n):
  z = jnp.zeros((_L,), jnp.float32)

  @functools.partial(plsc.parallel_loop, 0, n // _L, unroll=8)
  def _(i):
    ref[pl.ds(i * _L, _L)] = z


@functools.partial(
    pl.kernel,
    out_type=jax.ShapeDtypeStruct((NW, N), jnp.float32),
    mesh=_MESH,
    compiler_params=_SC_PARAMS,
    scratch_types=[
        pltpu.VMEM((EPW,), jnp.int32),
        pltpu.VMEM((EPW,), jnp.float32),
        pltpu.VMEM((N,), jnp.float32),
    ],
)
def _sc_degree(col_hbm, w_hbm, out_hbm, col_v, w_v, acc_v):
  wid = _worker_id()
  base = wid * EPW
  pltpu.sync_copy(col_hbm.at[pl.ds(base, EPW)], col_v)
  pltpu.sync_copy(w_hbm.at[pl.ds(base, EPW)], w_v)
  _zero_vmem(acc_v, N)

  @functools.partial(plsc.parallel_loop, 0, EPW // _L, unroll=4)
  def _(i):
    c = col_v[pl.ds(i * _L, _L)]
    wv = w_v[pl.ds(i * _L, _L)]
    plsc.addupdate_scatter(acc_v, [c], wv)

  pltpu.sync_copy(acc_v, out_hbm.at[wid])


@functools.partial(
    pl.kernel,
    out_type=jax.ShapeDtypeStruct((NW, N), jnp.float32),
    mesh=_MESH,
    compiler_params=_SC_PARAMS,
    scratch_types=[
        pltpu.VMEM((EPW,), jnp.int32),
        pltpu.VMEM((EPW,), jnp.int32),
        pltpu.VMEM((EPW,), jnp.float32),
        pltpu.VMEM((N,), jnp.float32),
        pltpu.VMEM((N,), jnp.float32),
    ],
)
def _sc_aggregate(row_hbm, col_hbm, w_hbm, g_hbm, out_hbm,
                  row_v, col_v, w_v, g_v, acc_v):
  wid = _worker_id()
  base = wid * EPW
  pltpu.sync_copy(row_hbm.at[pl.ds(base, EPW)], row_v)
  pltpu.sync_copy(col_hbm.at[pl.ds(base, EPW)], col_v)
  pltpu.sync_copy(w_hbm.at[pl.ds(base, EPW)], w_v)
  pltpu.sync_copy(g_hbm, g_v)
  _zero_vmem(acc_v, N)

  @functools.partial(plsc.parallel_loop, 0, EPW // _L, unroll=4)
  def _(i):
    r = row_v[pl.ds(i * _L, _L)]
    c = col_v[pl.ds(i * _L, _L)]
    wv = w_v[pl.ds(i * _L, _L)]
    gv = plsc.load_gather(g_v, [r])
    plsc.addupdate_scatter(acc_v, [c], wv * gv)

  pltpu.sync_copy(acc_v, out_hbm.at[wid])


def _tc_prep_body(x_ref, w_ref, parts_ref, dis_ref, g_ref):
  h_row = lax.dot_general(w_ref[...], x_ref[...],
                          dimension_numbers=(((0,), (1,)), ((), ())),
                          preferred_element_type=jnp.float32)
  deg = jnp.sum(parts_ref[...], axis=0, keepdims=True) + 1.0
  dis = lax.rsqrt(deg)
  dis_ref[...] = dis
  g_ref[...] = dis * h_row


def _tc_prep(x, w, parts):
  return pl.pallas_call(
      _tc_prep_body,
      out_shape=(
          jax.ShapeDtypeStruct((1, N), jnp.float32),
          jax.ShapeDtypeStruct((1, N), jnp.float32),
      ),
  )(x, w, parts)


def _tc_final_body(parts_ref, dis_ref, g_ref, b_ref, out_ref):
  s = jnp.sum(parts_ref[...], axis=0, keepdims=True)
  out_ref[...] = dis_ref[...] * (s + g_ref[...]) + b_ref[0, 0]


def _tc_final(parts, dis, g, b):
  return pl.pallas_call(
      _tc_final_body,
      out_shape=jax.ShapeDtypeStruct((1, N), jnp.float32),
  )(parts, dis, g, b.reshape(1, 1))


def kernel(x, edge_index, edge_weights, W, b):
  row = edge_index[0]
  col = edge_index[1]

  deg_parts = _sc_degree(col, edge_weights)
  dis, g = _tc_prep(x, W, deg_parts)
  s_parts = _sc_aggregate(row, col, edge_weights, g.reshape(N))
  out = _tc_final(s_parts, dis, g, b)
  return out.reshape(N, C)


# speedup vs baseline: 184.6579x; 1.2477x over previous
import functools

import jax
import jax.numpy as jnp
from jax import lax
from jax.experimental import pallas as pl
from jax.experimental.pallas import tpu as pltpu
from jax.experimental.pallas import tpu_sc as plsc

N = 10000
E = 320000
D = 128
C = 1

_INFO = plsc.get_sparse_core_info()
_NC = _INFO.num_cores
_NS = _INFO.num_subcores
NW = _NC * _NS
EPW = E // NW
_L = 16

_MESH = plsc.VectorSubcoreMesh(core_axis_name="c", subcore_axis_name="s")
_SC_PARAMS = pltpu.CompilerParams(needs_layout_passes=False)


def _worker_id():
  return lax.axis_index("s") * _NC + lax.axis_index("c")


def _zero_vmem(ref, n):
  z = jnp.zeros((_L,), jnp.float32)

  @functools.partial(plsc.parallel_loop, 0, n // _L, unroll=8)
  def _(i):
    ref[pl.ds(i * _L, _L)] = z


@functools.partial(
    pl.kernel,
    out_type=jax.ShapeDtypeStruct((NW, N), jnp.float32),
    mesh=_MESH,
    compiler_params=_SC_PARAMS,
    scratch_types=[
        pltpu.VMEM((EPW,), jnp.int32),
        pltpu.VMEM((EPW,), jnp.float32),
        pltpu.VMEM((N,), jnp.float32),
    ],
)
def _sc_degree(ei_hbm, w_hbm, out_hbm, col_v, w_v, acc_v):
  wid = _worker_id()
  base = wid * EPW
  pltpu.sync_copy(ei_hbm.at[pl.ds(E + base, EPW)], col_v)
  pltpu.sync_copy(w_hbm.at[pl.ds(base, EPW)], w_v)
  _zero_vmem(acc_v, N)

  @functools.partial(plsc.parallel_loop, 0, EPW // _L, unroll=4)
  def _(i):
    c = col_v[pl.ds(i * _L, _L)]
    wv = w_v[pl.ds(i * _L, _L)]
    plsc.addupdate_scatter(acc_v, [c], wv)

  pltpu.sync_copy(acc_v, out_hbm.at[wid])


@functools.partial(
    pl.kernel,
    out_type=jax.ShapeDtypeStruct((NW, N), jnp.float32),
    mesh=_MESH,
    compiler_params=_SC_PARAMS,
    scratch_types=[
        pltpu.VMEM((EPW,), jnp.int32),
        pltpu.VMEM((EPW,), jnp.int32),
        pltpu.VMEM((EPW,), jnp.float32),
        pltpu.VMEM((N,), jnp.float32),
        pltpu.VMEM((N,), jnp.float32),
    ],
)
def _sc_aggregate(ei_hbm, w_hbm, g_hbm, out_hbm,
                  row_v, col_v, w_v, g_v, acc_v):
  wid = _worker_id()
  base = wid * EPW
  pltpu.sync_copy(ei_hbm.at[pl.ds(base, EPW)], row_v)
  pltpu.sync_copy(ei_hbm.at[pl.ds(E + base, EPW)], col_v)
  pltpu.sync_copy(w_hbm.at[pl.ds(base, EPW)], w_v)
  pltpu.sync_copy(g_hbm, g_v)
  _zero_vmem(acc_v, N)

  @functools.partial(plsc.parallel_loop, 0, EPW // _L, unroll=4)
  def _(i):
    r = row_v[pl.ds(i * _L, _L)]
    c = col_v[pl.ds(i * _L, _L)]
    wv = w_v[pl.ds(i * _L, _L)]
    gv = plsc.load_gather(g_v, [r])
    plsc.addupdate_scatter(acc_v, [c], wv * gv)

  pltpu.sync_copy(acc_v, out_hbm.at[wid])


def _tc_prep_body(x_ref, w_ref, parts_ref, dis_ref, g_ref):
  h_row = lax.dot_general(w_ref[...], x_ref[...],
                          dimension_numbers=(((0,), (1,)), ((), ())),
                          preferred_element_type=jnp.float32)
  deg = jnp.sum(parts_ref[...], axis=0, keepdims=True) + 1.0
  dis = lax.rsqrt(deg)
  dis_ref[...] = dis
  g_ref[...] = dis * h_row


def _tc_prep(x, w, parts):
  return pl.pallas_call(
      _tc_prep_body,
      out_shape=(
          jax.ShapeDtypeStruct((1, N), jnp.float32),
          jax.ShapeDtypeStruct((1, N), jnp.float32),
      ),
  )(x, w, parts)


def _tc_final_body(parts_ref, dis_ref, g_ref, b_ref, out_ref):
  s = jnp.sum(parts_ref[...], axis=0, keepdims=True)
  out_ref[...] = dis_ref[...] * (s + g_ref[...]) + b_ref[0, 0]


def _tc_final(parts, dis, g, b):
  return pl.pallas_call(
      _tc_final_body,
      out_shape=jax.ShapeDtypeStruct((1, N), jnp.float32),
  )(parts, dis, g, b.reshape(1, 1))


def kernel(x, edge_index, edge_weights, W, b):
  ei_flat = edge_index.reshape(2 * E)
  deg_parts = _sc_degree(ei_flat, edge_weights)
  dis, g = _tc_prep(x, W, deg_parts)
  s_parts = _sc_aggregate(ei_flat, edge_weights, g.reshape(N))
  out = _tc_final(s_parts, dis, g, b)
  return out.reshape(N, C)


# speedup vs baseline: 198.8158x; 1.0767x over previous
import functools

import jax
import jax.numpy as jnp
from jax import lax
from jax.experimental import pallas as pl
from jax.experimental.pallas import tpu as pltpu
from jax.experimental.pallas import tpu_sc as plsc

N = 10000
E = 320000
D = 128
C = 1

_INFO = plsc.get_sparse_core_info()
_NC = _INFO.num_cores
_NS = _INFO.num_subcores
NW = _NC * _NS
_L = 16

_CH = 79 * 128
_EPW0 = 78 * 128

_MESH = plsc.VectorSubcoreMesh(core_axis_name="c", subcore_axis_name="s")
_SC_PARAMS = pltpu.CompilerParams(needs_layout_passes=False)


def _worker_id():
  return lax.axis_index("s") * _NC + lax.axis_index("c")


def _edge_range(wid):
  base = wid * _EPW0 + jnp.maximum(wid - 28, 0) * 128
  steps = jnp.where(wid >= 28, _CH // _L, _EPW0 // _L)
  return pl.multiple_of(base, 128), steps


def _zero_vmem(ref, n):
  z = jnp.zeros((_L,), jnp.float32)

  @functools.partial(plsc.parallel_loop, 0, n // _L, unroll=8)
  def _(i):
    ref[pl.ds(i * _L, _L)] = z


@functools.partial(
    pl.kernel,
    out_type=jax.ShapeDtypeStruct((NW, N), jnp.float32),
    mesh=_MESH,
    compiler_params=_SC_PARAMS,
    scratch_types=[
        pltpu.VMEM((2, _CH), jnp.int32),
        pltpu.VMEM((_CH,), jnp.float32),
        pltpu.VMEM((N,), jnp.float32),
    ],
)
def _sc_degree(ei_hbm, w_hbm, out_hbm, ei_v, w_v, acc_v):
  wid = _worker_id()
  base, steps = _edge_range(wid)
  pltpu.sync_copy(ei_hbm.at[:, pl.ds(base, _CH)], ei_v)
  pltpu.sync_copy(w_hbm.at[pl.ds(base, _CH)], w_v)
  _zero_vmem(acc_v, N)

  @functools.partial(plsc.parallel_loop, 0, steps, unroll=4)
  def _(i):
    c = ei_v[1, pl.ds(i * _L, _L)]
    wv = w_v[pl.ds(i * _L, _L)]
    plsc.addupdate_scatter(acc_v, [c], wv)

  pltpu.sync_copy(acc_v, out_hbm.at[wid])


@functools.partial(
    pl.kernel,
    out_type=jax.ShapeDtypeStruct((NW, N), jnp.float32),
    mesh=_MESH,
    compiler_params=_SC_PARAMS,
    scratch_types=[
        pltpu.VMEM((2, _CH), jnp.int32),
        pltpu.VMEM((_CH,), jnp.float32),
        pltpu.VMEM((N,), jnp.float32),
        pltpu.VMEM((N,), jnp.float32),
    ],
)
def _sc_aggregate(ei_hbm, w_hbm, g_hbm, out_hbm,
                  ei_v, w_v, g_v, acc_v):
  wid = _worker_id()
  base, steps = _edge_range(wid)
  pltpu.sync_copy(ei_hbm.at[:, pl.ds(base, _CH)], ei_v)
  pltpu.sync_copy(w_hbm.at[pl.ds(base, _CH)], w_v)
  pltpu.sync_copy(g_hbm, g_v)
  _zero_vmem(acc_v, N)

  @functools.partial(plsc.parallel_loop, 0, steps, unroll=4)
  def _(i):
    r = ei_v[0, pl.ds(i * _L, _L)]
    c = ei_v[1, pl.ds(i * _L, _L)]
    wv = w_v[pl.ds(i * _L, _L)]
    gv = plsc.load_gather(g_v, [r])
    plsc.addupdate_scatter(acc_v, [c], wv * gv)

  pltpu.sync_copy(acc_v, out_hbm.at[wid])


def _tc_prep_body(x_ref, w_ref, parts_ref, dis_ref, g_ref):
  h_row = lax.dot_general(w_ref[...], x_ref[...],
                          dimension_numbers=(((0,), (1,)), ((), ())),
                          preferred_element_type=jnp.float32)
  deg = jnp.sum(parts_ref[...], axis=0, keepdims=True) + 1.0
  dis = lax.rsqrt(deg)
  dis_ref[...] = dis
  g_ref[...] = dis * h_row


def _tc_prep(x, w, parts):
  return pl.pallas_call(
      _tc_prep_body,
      out_shape=(
          jax.ShapeDtypeStruct((1, N), jnp.float32),
          jax.ShapeDtypeStruct((1, N), jnp.float32),
      ),
  )(x, w, parts)


def _tc_final_body(parts_ref, dis_ref, g_ref, b_ref, out_ref):
  s = jnp.sum(parts_ref[...], axis=0, keepdims=True)
  out_ref[...] = dis_ref[...] * (s + g_ref[...]) + b_ref[0, 0]


def _tc_final(parts, dis, g, b):
  return pl.pallas_call(
      _tc_final_body,
      out_shape=jax.ShapeDtypeStruct((1, N), jnp.float32),
  )(parts, dis, g, b.reshape(1, 1))


def kernel(x, edge_index, edge_weights, W, b):
  deg_parts = _sc_degree(edge_index, edge_weights)
  dis, g = _tc_prep(x, W, deg_parts)
  s_parts = _sc_aggregate(edge_index, edge_weights, g.reshape(N))
  out = _tc_final(s_parts, dis, g, b)
  return out.reshape(N, C)


# speedup vs baseline: 207.6764x; 1.0446x over previous
import functools

import jax
import jax.numpy as jnp
from jax import lax
from jax.experimental import pallas as pl
from jax.experimental.pallas import tpu as pltpu
from jax.experimental.pallas import tpu_sc as plsc

N = 10000
E = 320000
D = 128
C = 1

_INFO = plsc.get_sparse_core_info()
_NC = _INFO.num_cores
_NS = _INFO.num_subcores
NW = _NC * _NS
_L = 16

_CH = 79 * 128
_EPW0 = 78 * 128

_MESH = plsc.VectorSubcoreMesh(core_axis_name="c", subcore_axis_name="s")
_SC_PARAMS = pltpu.CompilerParams(needs_layout_passes=False)


def _worker_id():
  return lax.axis_index("s") * _NC + lax.axis_index("c")


def _edge_range(wid):
  base = wid * _EPW0 + jnp.maximum(wid - 28, 0) * 128
  steps = jnp.where(wid >= 28, _CH // _L, _EPW0 // _L)
  return pl.multiple_of(base, 128), steps


def _zero_vmem(ref, n):
  z = jnp.zeros((_L,), jnp.float32)

  @functools.partial(plsc.parallel_loop, 0, n // _L, unroll=8)
  def _(i):
    ref[pl.ds(i * _L, _L)] = z


@functools.partial(
    pl.kernel,
    out_type=jax.ShapeDtypeStruct((NW, N), jnp.float32),
    mesh=_MESH,
    compiler_params=_SC_PARAMS,
    scratch_types=[
        pltpu.VMEM((2, _CH), jnp.int32),
        pltpu.VMEM((_CH,), jnp.float32),
        pltpu.VMEM((N,), jnp.float32),
    ],
)
def _sc_degree(ei_hbm, w_hbm, out_hbm, ei_v, w_v, acc_v):
  wid = _worker_id()
  base, steps = _edge_range(wid)
  pltpu.sync_copy(ei_hbm.at[:, pl.ds(base, _CH)], ei_v)
  pltpu.sync_copy(w_hbm.at[pl.ds(base, _CH)], w_v)
  _zero_vmem(acc_v, N)

  @functools.partial(plsc.parallel_loop, 0, steps, unroll=4)
  def _(i):
    c = ei_v[1, pl.ds(i * _L, _L)]
    wv = w_v[pl.ds(i * _L, _L)]
    plsc.addupdate_scatter(acc_v, [c], wv)

  pltpu.sync_copy(acc_v, out_hbm.at[wid])


@functools.partial(
    pl.kernel,
    out_type=jax.ShapeDtypeStruct((NW, N), jnp.float32),
    mesh=_MESH,
    compiler_params=_SC_PARAMS,
    scratch_types=[
        pltpu.VMEM((2, _CH), jnp.int32),
        pltpu.VMEM((_CH,), jnp.float32),
        pltpu.VMEM((N,), jnp.float32),
        pltpu.VMEM((N,), jnp.float32),
    ],
)
def _sc_aggregate(ei_hbm, w_hbm, g_hbm, out_hbm,
                  ei_v, w_v, g_v, acc_v):
  wid = _worker_id()
  base, steps = _edge_range(wid)
  pltpu.sync_copy(ei_hbm.at[:, pl.ds(base, _CH)], ei_v)
  pltpu.sync_copy(w_hbm.at[pl.ds(base, _CH)], w_v)
  pltpu.sync_copy(g_hbm, g_v)
  _zero_vmem(acc_v, N)

  @functools.partial(plsc.parallel_loop, 0, steps, unroll=4)
  def _(i):
    r = ei_v[0, pl.ds(i * _L, _L)]
    c = ei_v[1, pl.ds(i * _L, _L)]
    wv = w_v[pl.ds(i * _L, _L)]
    gv = plsc.load_gather(g_v, [r])
    plsc.addupdate_scatter(acc_v, [c], wv * gv)

  pltpu.sync_copy(acc_v, out_hbm.at[wid])


def _tc_matvec_body(x_ref, w_ref, h_ref):
  h_ref[...] = lax.dot_general(w_ref[...], x_ref[...],
                               dimension_numbers=(((0,), (1,)), ((), ())),
                               preferred_element_type=jnp.float32)


def _tc_matvec(x, w):
  return pl.pallas_call(
      _tc_matvec_body,
      out_shape=jax.ShapeDtypeStruct((1, N), jnp.float32),
  )(x, w)


def _tc_prep_body(parts_ref, h_ref, dis_ref, g_ref):
  deg = jnp.sum(parts_ref[...], axis=0, keepdims=True) + 1.0
  dis = lax.rsqrt(deg)
  dis_ref[...] = dis
  g_ref[...] = dis * h_ref[...]


def _tc_prep(parts, h_row):
  return pl.pallas_call(
      _tc_prep_body,
      out_shape=(
          jax.ShapeDtypeStruct((1, N), jnp.float32),
          jax.ShapeDtypeStruct((1, N), jnp.float32),
      ),
  )(parts, h_row)


def _tc_final_body(parts_ref, dis_ref, g_ref, b_ref, out_ref):
  s = jnp.sum(parts_ref[...], axis=0, keepdims=True)
  out_ref[...] = dis_ref[...] * (s + g_ref[...]) + b_ref[0, 0]


def _tc_final(parts, dis, g, b):
  return pl.pallas_call(
      _tc_final_body,
      out_shape=jax.ShapeDtypeStruct((1, N), jnp.float32),
  )(parts, dis, g, b.reshape(1, 1))


def kernel(x, edge_index, edge_weights, W, b):
  deg_parts = _sc_degree(edge_index, edge_weights)
  h_row = _tc_matvec(x, W)
  dis, g = _tc_prep(deg_parts, h_row)
  s_parts = _sc_aggregate(edge_index, edge_weights, g.reshape(N))
  out = _tc_final(s_parts, dis, g, b)
  return out.reshape(N, C)


# speedup vs baseline: 208.2756x; 1.0029x over previous
import functools

import jax
import jax.numpy as jnp
from jax import lax
from jax.experimental import pallas as pl
from jax.experimental.pallas import tpu as pltpu
from jax.experimental.pallas import tpu_sc as plsc

N = 10000
E = 320000
D = 128
C = 1

_INFO = plsc.get_sparse_core_info()
_NC = _INFO.num_cores
_NS = _INFO.num_subcores
NW = _NC * _NS
_L = 16

_CH = 79 * 128
_EPW0 = 78 * 128

_MESH = plsc.VectorSubcoreMesh(core_axis_name="c", subcore_axis_name="s")
_SC_PARAMS = pltpu.CompilerParams(needs_layout_passes=False)


def _worker_id():
  return lax.axis_index("s") * _NC + lax.axis_index("c")


def _edge_range(wid):
  base = wid * _EPW0 + jnp.maximum(wid - 28, 0) * 128
  steps = jnp.where(wid >= 28, _CH // _L, _EPW0 // _L)
  return pl.multiple_of(base, 128), steps


def _zero_vmem(ref, n):
  z = jnp.zeros((_L,), jnp.float32)

  @functools.partial(plsc.parallel_loop, 0, n // _L, unroll=8)
  def _(i):
    ref[pl.ds(i * _L, _L)] = z


@functools.partial(
    pl.kernel,
    out_type=jax.ShapeDtypeStruct((NW, N), jnp.float32),
    mesh=_MESH,
    compiler_params=_SC_PARAMS,
    scratch_types=[
        pltpu.VMEM((2, _CH), jnp.int32),
        pltpu.VMEM((_CH,), jnp.float32),
        pltpu.VMEM((N,), jnp.float32),
    ],
)
def _sc_degree(ei_hbm, w_hbm, out_hbm, ei_v, w_v, acc_v):
  wid = _worker_id()
  base, steps = _edge_range(wid)
  pltpu.sync_copy(ei_hbm.at[:, pl.ds(base, _CH)], ei_v)
  pltpu.sync_copy(w_hbm.at[pl.ds(base, _CH)], w_v)
  _zero_vmem(acc_v, N)

  @functools.partial(plsc.parallel_loop, 0, steps, unroll=8)
  def _(i):
    c = ei_v[1, pl.ds(i * _L, _L)]
    wv = w_v[pl.ds(i * _L, _L)]
    plsc.addupdate_scatter(acc_v, [c], wv)

  pltpu.sync_copy(acc_v, out_hbm.at[wid])


@functools.partial(
    pl.kernel,
    out_type=jax.ShapeDtypeStruct((NW, N), jnp.float32),
    mesh=_MESH,
    compiler_params=_SC_PARAMS,
    scratch_types=[
        pltpu.VMEM((2, _CH), jnp.int32),
        pltpu.VMEM((_CH,), jnp.float32),
        pltpu.VMEM((N,), jnp.float32),
        pltpu.VMEM((N,), jnp.float32),
    ],
)
def _sc_aggregate(ei_hbm, w_hbm, g_hbm, out_hbm,
                  ei_v, w_v, g_v, acc_v):
  wid = _worker_id()
  base, steps = _edge_range(wid)
  pltpu.sync_copy(ei_hbm.at[:, pl.ds(base, _CH)], ei_v)
  pltpu.sync_copy(w_hbm.at[pl.ds(base, _CH)], w_v)
  pltpu.sync_copy(g_hbm, g_v)
  _zero_vmem(acc_v, N)

  @functools.partial(plsc.parallel_loop, 0, steps, unroll=8)
  def _(i):
    r = ei_v[0, pl.ds(i * _L, _L)]
    c = ei_v[1, pl.ds(i * _L, _L)]
    wv = w_v[pl.ds(i * _L, _L)]
    gv = plsc.load_gather(g_v, [r])
    plsc.addupdate_scatter(acc_v, [c], wv * gv)

  pltpu.sync_copy(acc_v, out_hbm.at[wid])


def _tc_matvec_body(x_ref, w_ref, h_ref):
  h_ref[...] = lax.dot_general(w_ref[...], x_ref[...],
                               dimension_numbers=(((0,), (1,)), ((), ())),
                               preferred_element_type=jnp.float32)


def _tc_matvec(x, w):
  return pl.pallas_call(
      _tc_matvec_body,
      out_shape=jax.ShapeDtypeStruct((1, N), jnp.float32),
  )(x, w)


def _tc_prep_body(parts_ref, h_ref, dis_ref, g_ref):
  deg = jnp.sum(parts_ref[...], axis=0, keepdims=True) + 1.0
  dis = lax.rsqrt(deg)
  dis_ref[...] = dis
  g_ref[...] = dis * h_ref[...]


def _tc_prep(parts, h_row):
  return pl.pallas_call(
      _tc_prep_body,
      out_shape=(
          jax.ShapeDtypeStruct((1, N), jnp.float32),
          jax.ShapeDtypeStruct((1, N), jnp.float32),
      ),
  )(parts, h_row)


def _tc_final_body(parts_ref, dis_ref, g_ref, b_ref, out_ref):
  s = jnp.sum(parts_ref[...], axis=0, keepdims=True)
  out_ref[...] = dis_ref[...] * (s + g_ref[...]) + b_ref[0, 0]


def _tc_final(parts, dis, g, b):
  return pl.pallas_call(
      _tc_final_body,
      out_shape=jax.ShapeDtypeStruct((1, N), jnp.float32),
  )(parts, dis, g, b.reshape(1, 1))


def kernel(x, edge_index, edge_weights, W, b):
  deg_parts = _sc_degree(edge_index, edge_weights)
  h_row = _tc_matvec(x, W)
  dis, g = _tc_prep(deg_parts, h_row)
  s_parts = _sc_aggregate(edge_index, edge_weights, g.reshape(N))
  out = _tc_final(s_parts, dis, g, b)
  return out.reshape(N, C)
